# baseline (speedup 1.0000x reference)
import functools

import numpy as np
import jax
import jax.numpy as jnp
from jax.experimental import pallas as pl
from jax.experimental.pallas import tpu as pltpu
from jax.experimental.pallas import tpu_sc as plsc

F = 26
B = 16384
VOCAB = 100000
D = 32
H1, H2 = 512, 256
FD = F * D
FF = F * F
GW = 128
BBLK = 512


def _sc_gather(tables, gidx):
    mesh = plsc.VectorSubcoreMesh(core_axis_name="core", subcore_axis_name="subcore")

    n_windows = B // GW
    n_tasks = F * n_windows
    n_workers = 32
    tpw = n_tasks // n_workers

    @functools.partial(
        pl.kernel,
        out_type=jax.ShapeDtypeStruct((F, B // 4, 4 * D), jnp.float32),
        mesh=mesh,
        scratch_types=[
            pltpu.VMEM((GW,), jnp.int32),
            pltpu.VMEM((GW, D), jnp.float32),
            pltpu.SemaphoreType.DMA,
        ],
        compiler_params=pltpu.CompilerParams(use_tc_tiling_on_sc=False),
    )
    def gather_kernel(tab_hbm, idx_hbm, out_hbm, idx_v, rows_v, sem):
        wid = jax.lax.axis_index("subcore") * 2 + jax.lax.axis_index("core")

        @pl.loop(0, tpw)
        def _(t):
            task = wid * tpw + t
            f = task // n_windows
            w = task % n_windows
            pltpu.sync_copy(idx_hbm.at[task], idx_v)
            pltpu.async_copy(tab_hbm.at[f].at[idx_v], rows_v, sem).wait()
            for s in range(4):
                pltpu.sync_copy(
                    rows_v.at[pl.ds(s * (GW // 4), GW // 4)],
                    out_hbm.at[f, pl.ds(w * (GW // 4), GW // 4), pl.ds(s * D, D)],
                )

    return gather_kernel(tables, gidx)


def _tc_body(embs_ref, w1a_ref, w1c_ref, b1_ref, w2_ref, b2_ref, w3_ref, b3_ref,
             out_ref, dots_ref):
    e4 = embs_ref[...]
    e4t = jnp.transpose(e4, (0, 2, 1))
    e4r = e4t.reshape(F, 4, D, BBLK // 4)
    ft3 = jnp.concatenate([e4r[:, s] for s in range(4)], axis=-1)
    ft = ft3.reshape(FD, BBLK)
    for i in range(F):
        prod = ft3 * ft3[i][None]
        dots_ref[pl.ds(i * F, F), :] = jnp.sum(prod, axis=1)
    h = jnp.dot(w1a_ref[...], ft, preferred_element_type=jnp.float32)
    h = h + jnp.dot(w1c_ref[...], dots_ref[...], preferred_element_type=jnp.float32)
    h = jnp.maximum(h + b1_ref[...], 0.0)
    h2 = jnp.dot(w2_ref[...], h, preferred_element_type=jnp.float32) + b2_ref[...]
    h2 = jnp.maximum(h2, 0.0)
    out_ref[...] = jnp.dot(w3_ref[...], h2, preferred_element_type=jnp.float32) + b3_ref[...]


def _tc_mlp(embs, w1aT, w1cT, b1c, w2T, b2c, w3T, b3s):
    return pl.pallas_call(
        _tc_body,
        grid=(B // BBLK,),
        in_specs=[
            pl.BlockSpec((F, BBLK // 4, 4 * D), lambda i: (0, i, 0)),
            pl.BlockSpec((H1, FD), lambda i: (0, 0)),
            pl.BlockSpec((H1, FF), lambda i: (0, 0)),
            pl.BlockSpec((H1, 1), lambda i: (0, 0)),
            pl.BlockSpec((H2, H1), lambda i: (0, 0)),
            pl.BlockSpec((H2, 1), lambda i: (0, 0)),
            pl.BlockSpec((1, H2), lambda i: (0, 0)),
            pl.BlockSpec((1, 1), lambda i: (0, 0)),
        ],
        out_specs=pl.BlockSpec((1, BBLK), lambda i: (0, i)),
        out_shape=jax.ShapeDtypeStruct((1, B), jnp.float32),
        scratch_shapes=[pltpu.VMEM((FF, BBLK), jnp.float32)],
    )(embs, w1aT, w1cT, b1c, w2T, b2c, w3T, b3s)


_IU, _JU = np.triu_indices(F, k=1)
_UP = np.asarray(_IU * F + _JU)
_LO = np.asarray(_JU * F + _IU)


def kernel(indices, tables, W1, b1, W2, b2, W3, b3):
    gidx = indices.reshape(F * B // GW, GW)
    embs = _sc_gather(tables, gidx)

    W1a = W1[:FD]
    W1b = 0.5 * W1[FD:]
    W1c = jnp.zeros((FF, H1), W1.dtype).at[_UP].set(W1b).at[_LO].set(W1b)
    out2d = _tc_mlp(embs, W1a.T, W1c.T, b1[:, None], W2.T, b2[:, None],
                    W3.T, b3.reshape(1, 1))
    outp = out2d.reshape(B // BBLK, 4, 4, 32)
    return jnp.transpose(outp, (0, 2, 1, 3)).reshape(B)


# speedup vs baseline: 1.0086x; 1.0086x over previous
import functools

import numpy as np
import jax
import jax.numpy as jnp
from jax.experimental import pallas as pl
from jax.experimental.pallas import tpu as pltpu
from jax.experimental.pallas import tpu_sc as plsc

F = 26
B = 16384
VOCAB = 100000
D = 32
H1, H2 = 512, 256
FD = F * D
FF = F * F
GW = 128
BBLK = 512


def _sc_gather(tables2d, gidx):
    mesh = plsc.VectorSubcoreMesh(core_axis_name="core", subcore_axis_name="subcore")

    @functools.partial(
        pl.kernel,
        out_type=jax.ShapeDtypeStruct((B, FD), jnp.float32),
        mesh=mesh,
        compiler_params=pltpu.CompilerParams(use_tc_tiling_on_sc=False),
    )
    def gather_kernel(tab_hbm, idx_hbm, out_hbm):
        def body(i_vmem, o_vmem):
            pltpu.sync_copy(tab_hbm.at[i_vmem.at[0]], o_vmem)

        pltpu.emit_pipeline(
            body,
            grid=(F, B // GW),
            in_specs=[
                pl.BlockSpec((1, GW), index_map=lambda f, b: (0, f * (B // GW) + b))
            ],
            out_specs=[pl.BlockSpec((GW, D), index_map=lambda f, b: (b, f))],
            core_axis_name=("core", "subcore"),
            dimension_semantics=(pltpu.PARALLEL, pltpu.PARALLEL),
        )(idx_hbm, out_hbm)

    return gather_kernel(tables2d, gidx)


def _tc_body(flat_ref, w1a_ref, w1c_ref, b1_ref, w2_ref, b2_ref, w3_ref, b3_ref,
             out_ref, dots_ref):
    fl = flat_ref[...]
    ft = fl.T
    ft3 = ft.reshape(F, D, BBLK)
    for i in range(F):
        prod = ft3 * ft3[i][None]
        dots_ref[pl.ds(i * F, F), :] = jnp.sum(prod, axis=1)
    bf = jnp.bfloat16
    h = jnp.dot(w1a_ref[...].astype(bf), ft.astype(bf),
                preferred_element_type=jnp.float32)
    h = h + jnp.dot(w1c_ref[...].astype(bf), dots_ref[...].astype(bf),
                    preferred_element_type=jnp.float32)
    h = jnp.maximum(h + b1_ref[...], 0.0)
    h2 = jnp.dot(w2_ref[...].astype(bf), h.astype(bf),
                 preferred_element_type=jnp.float32) + b2_ref[...]
    h2 = jnp.maximum(h2, 0.0)
    out_ref[...] = jnp.dot(w3_ref[...], h2, preferred_element_type=jnp.float32) + b3_ref[...]


def _tc_mlp(embs, w1aT, w1cT, b1c, w2T, b2c, w3T, b3s):
    return pl.pallas_call(
        _tc_body,
        grid=(B // BBLK,),
        in_specs=[
            pl.BlockSpec((BBLK, FD), lambda i: (i, 0)),
            pl.BlockSpec((H1, FD), lambda i: (0, 0)),
            pl.BlockSpec((H1, FF), lambda i: (0, 0)),
            pl.BlockSpec((H1, 1), lambda i: (0, 0)),
            pl.BlockSpec((H2, H1), lambda i: (0, 0)),
            pl.BlockSpec((H2, 1), lambda i: (0, 0)),
            pl.BlockSpec((1, H2), lambda i: (0, 0)),
            pl.BlockSpec((1, 1), lambda i: (0, 0)),
        ],
        out_specs=pl.BlockSpec((1, BBLK), lambda i: (0, i)),
        out_shape=jax.ShapeDtypeStruct((1, B), jnp.float32),
        scratch_shapes=[pltpu.VMEM((FF, BBLK), jnp.float32)],
    )(embs, w1aT, w1cT, b1c, w2T, b2c, w3T, b3s)


_IU, _JU = np.triu_indices(F, k=1)
_UP = np.asarray(_IU * F + _JU)
_LO = np.asarray(_JU * F + _IU)


def kernel(indices, tables, W1, b1, W2, b2, W3, b3):
    tables2d = tables.reshape(F * VOCAB, D)
    offs = (jnp.arange(F, dtype=jnp.int32) * VOCAB)[:, None]
    gidx = (indices + offs).reshape(1, F * B)
    embs = _sc_gather(tables2d, gidx)

    W1a = W1[:FD]
    W1b = 0.5 * W1[FD:]
    W1c = jnp.zeros((FF, H1), W1.dtype).at[_UP].set(W1b).at[_LO].set(W1b)
    out2d = _tc_mlp(embs, W1a.T, W1c.T, b1[:, None], W2.T, b2[:, None],
                    W3.T, b3.reshape(1, 1))
    return out2d.reshape(B)


# speedup vs baseline: 1.0117x; 1.0030x over previous
import functools

import numpy as np
import jax
import jax.numpy as jnp
from jax.experimental import pallas as pl
from jax.experimental.pallas import tpu as pltpu
from jax.experimental.pallas import tpu_sc as plsc

F = 26
B = 16384
VOCAB = 100000
D = 32
H1, H2 = 512, 256
FD = F * D
FF = F * F
GW = 128
BBLK = 512


def _sc_gather(tables2d, gidx):
    mesh = plsc.VectorSubcoreMesh(core_axis_name="core", subcore_axis_name="subcore")

    @functools.partial(
        pl.kernel,
        out_type=jax.ShapeDtypeStruct((B, FD), jnp.float32),
        mesh=mesh,
        compiler_params=pltpu.CompilerParams(use_tc_tiling_on_sc=False),
    )
    def gather_kernel(tab_hbm, idx_hbm, out_hbm):
        def body(i_vmem, o_vmem):
            pltpu.sync_copy(tab_hbm.at[i_vmem.at[0]], o_vmem)

        pltpu.emit_pipeline(
            body,
            grid=(F, B // GW),
            in_specs=[
                pl.BlockSpec((1, GW), index_map=lambda f, b: (0, f * (B // GW) + b))
            ],
            out_specs=[pl.BlockSpec((GW, D), index_map=lambda f, b: (b, f))],
            core_axis_name=("core", "subcore"),
            dimension_semantics=(pltpu.PARALLEL, pltpu.PARALLEL),
        )(idx_hbm, out_hbm)

    return gather_kernel(tables2d, gidx)


def _tc_body(flat_ref, w1a_ref, w1c_ref, b1_ref, w2_ref, b2_ref, w3_ref, b3_ref,
             out_ref, dots_ref):
    fl = flat_ref[...]
    ft = fl.T
    ft3 = ft.reshape(F, D, BBLK)
    for i in range(F):
        prod = ft3 * ft3[i][None]
        dots_ref[pl.ds(i * F, F), :] = jnp.sum(prod, axis=1)
    h = jnp.dot(w1a_ref[...], ft, preferred_element_type=jnp.float32)
    h = h + jnp.dot(w1c_ref[...], dots_ref[...], preferred_element_type=jnp.float32)
    h = jnp.maximum(h + b1_ref[...], 0.0)
    h2 = jnp.dot(w2_ref[...], h, preferred_element_type=jnp.float32) + b2_ref[...]
    h2 = jnp.maximum(h2, 0.0)
    out_ref[...] = jnp.dot(w3_ref[...], h2, preferred_element_type=jnp.float32) + b3_ref[...]


def _tc_mlp(embs, w1aT, w1cT, b1c, w2T, b2c, w3T, b3s):
    return pl.pallas_call(
        _tc_body,
        grid=(B // BBLK,),
        in_specs=[
            pl.BlockSpec((BBLK, FD), lambda i: (i, 0)),
            pl.BlockSpec((H1, FD), lambda i: (0, 0)),
            pl.BlockSpec((H1, FF), lambda i: (0, 0)),
            pl.BlockSpec((H1, 1), lambda i: (0, 0)),
            pl.BlockSpec((H2, H1), lambda i: (0, 0)),
            pl.BlockSpec((H2, 1), lambda i: (0, 0)),
            pl.BlockSpec((1, H2), lambda i: (0, 0)),
            pl.BlockSpec((1, 1), lambda i: (0, 0)),
        ],
        out_specs=pl.BlockSpec((1, BBLK), lambda i: (0, i)),
        out_shape=jax.ShapeDtypeStruct((1, B), jnp.float32),
        scratch_shapes=[pltpu.VMEM((FF, BBLK), jnp.float32)],
    )(embs, w1aT, w1cT, b1c, w2T, b2c, w3T, b3s)


_IU, _JU = np.triu_indices(F, k=1)
_UP = np.asarray(_IU * F + _JU)
_LO = np.asarray(_JU * F + _IU)


def kernel(indices, tables, W1, b1, W2, b2, W3, b3):
    tables2d = tables.reshape(F * VOCAB, D)
    offs = (jnp.arange(F, dtype=jnp.int32) * VOCAB)[:, None]
    gidx = (indices + offs).reshape(1, F * B)
    embs = _sc_gather(tables2d, gidx)

    W1a = W1[:FD]
    W1b = 0.5 * W1[FD:]
    W1c = jnp.zeros((FF, H1), W1.dtype).at[_UP].set(W1b).at[_LO].set(W1b)
    out2d = _tc_mlp(embs, W1a.T, W1c.T, b1[:, None], W2.T, b2[:, None],
                    W3.T, b3.reshape(1, 1))
    return out2d.reshape(B)


# speedup vs baseline: 1.0220x; 1.0101x over previous
import functools

import numpy as np
import jax
import jax.numpy as jnp
from jax.experimental import pallas as pl
from jax.experimental.pallas import tpu as pltpu
from jax.experimental.pallas import tpu_sc as plsc

F = 26
B = 16384
VOCAB = 100000
D = 32
H1, H2 = 512, 256
FD = F * D
FF = F * F
GW = 128
BBLK = 512


def _sc_gather(tables2d, gidx, bc):
    mesh = plsc.VectorSubcoreMesh(core_axis_name="core", subcore_axis_name="subcore")

    @functools.partial(
        pl.kernel,
        out_type=jax.ShapeDtypeStruct((bc, FD), jnp.float32),
        mesh=mesh,
        compiler_params=pltpu.CompilerParams(use_tc_tiling_on_sc=False),
    )
    def gather_kernel(tab_hbm, idx_hbm, out_hbm):
        def body(i_vmem, o_vmem):
            pltpu.sync_copy(tab_hbm.at[i_vmem.at[0]], o_vmem)

        pltpu.emit_pipeline(
            body,
            grid=(F, bc // GW),
            in_specs=[
                pl.BlockSpec((1, GW), index_map=lambda f, b: (0, f * (bc // GW) + b))
            ],
            out_specs=[pl.BlockSpec((GW, D), index_map=lambda f, b: (b, f))],
            core_axis_name=("core", "subcore"),
            dimension_semantics=(pltpu.PARALLEL, pltpu.PARALLEL),
        )(idx_hbm, out_hbm)

    return gather_kernel(tables2d, gidx)


def _tc_body(flat_ref, w1a_ref, w1c_ref, b1_ref, w2_ref, b2_ref, w3_ref, b3_ref,
             out_ref, dots_ref):
    fl = flat_ref[...]
    ft = fl.T
    ft3 = ft.reshape(F, D, BBLK)
    for i in range(F):
        prod = ft3 * ft3[i][None]
        dots_ref[pl.ds(i * F, F), :] = jnp.sum(prod, axis=1)
    h = jnp.dot(w1a_ref[...], ft, preferred_element_type=jnp.float32)
    h = h + jnp.dot(w1c_ref[...], dots_ref[...], preferred_element_type=jnp.float32)
    h = jnp.maximum(h + b1_ref[...], 0.0)
    h2 = jnp.dot(w2_ref[...], h, preferred_element_type=jnp.float32) + b2_ref[...]
    h2 = jnp.maximum(h2, 0.0)
    out_ref[...] = jnp.dot(w3_ref[...], h2, preferred_element_type=jnp.float32) + b3_ref[...]


def _tc_mlp(embs, w1aT, w1cT, b1c, w2T, b2c, w3T, b3s, bc):
    return pl.pallas_call(
        _tc_body,
        grid=(bc // BBLK,),
        in_specs=[
            pl.BlockSpec((BBLK, FD), lambda i: (i, 0)),
            pl.BlockSpec((H1, FD), lambda i: (0, 0)),
            pl.BlockSpec((H1, FF), lambda i: (0, 0)),
            pl.BlockSpec((H1, 1), lambda i: (0, 0)),
            pl.BlockSpec((H2, H1), lambda i: (0, 0)),
            pl.BlockSpec((H2, 1), lambda i: (0, 0)),
            pl.BlockSpec((1, H2), lambda i: (0, 0)),
            pl.BlockSpec((1, 1), lambda i: (0, 0)),
        ],
        out_specs=pl.BlockSpec((1, BBLK), lambda i: (0, i)),
        out_shape=jax.ShapeDtypeStruct((1, bc), jnp.float32),
        scratch_shapes=[pltpu.VMEM((FF, BBLK), jnp.float32)],
    )(embs, w1aT, w1cT, b1c, w2T, b2c, w3T, b3s)


_IU, _JU = np.triu_indices(F, k=1)
_UP = np.asarray(_IU * F + _JU)
_LO = np.asarray(_JU * F + _IU)


NCHUNK = 2


def kernel(indices, tables, W1, b1, W2, b2, W3, b3):
    tables2d = tables.reshape(F * VOCAB, D)
    offs = (jnp.arange(F, dtype=jnp.int32) * VOCAB)[:, None]
    gall = indices + offs

    W1a = W1[:FD]
    W1b = 0.5 * W1[FD:]
    W1c = jnp.zeros((FF, H1), W1.dtype).at[_UP].set(W1b).at[_LO].set(W1b)

    bc = B // NCHUNK
    outs = []
    for c in range(NCHUNK):
        gidx = gall[:, c * bc:(c + 1) * bc].reshape(1, F * bc)
        flat = _sc_gather(tables2d, gidx, bc)
        outs.append(_tc_mlp(flat, W1a.T, W1c.T, b1[:, None], W2.T, b2[:, None],
                            W3.T, b3.reshape(1, 1), bc))
    return jnp.concatenate(outs, axis=1).reshape(B)
